# baseline (speedup 1.0000x reference)
import jax
import jax.numpy as jnp
from jax import lax
from jax.experimental import pallas as pl
from jax.experimental.pallas import tpu as pltpu
from jax.experimental.pallas import tpu_sc as plsc

N_TOK = 32768
N_EXP = 64
KK = 8
NUM_CORES = 2
NUM_SUBCORES = 16
NW = NUM_CORES * NUM_SUBCORES
TOK_PER_W = N_TOK // NW
TCHUNK = 256
NCHUNK = TOK_PER_W // TCHUNK
NGROUP = TCHUNK // 16
NEG = -3.0e38


def _body(x_hbm, o_hbm, xb0, xb1, ob0, ob1, si0, si1, so0, so1):
    wid = lax.axis_index("s") * NUM_CORES + lax.axis_index("c")
    base = wid * TOK_PER_W
    xbufs, obufs, sins, souts = (xb0, xb1), (ob0, ob1), (si0, si1), (so0, so1)

    def start_in(c, b):
        return pltpu.async_copy(
            x_hbm.at[:, pl.ds(base + c * TCHUNK, TCHUNK)], xbufs[b], sins[b]
        )

    def compute_chunk(xbuf, obuf):
        @plsc.parallel_loop(0, NGROUP, step=1)
        def group_body(g):
            t0 = g * 16
            neg = jnp.full((16,), NEG, jnp.float32)
            m = [neg] * KK
            for j in range(N_EXP):
                c = xbuf[j, pl.ds(t0, 16)]
                for i in range(KK):
                    hi = jnp.maximum(m[i], c)
                    c = jnp.minimum(m[i], c)
                    m[i] = hi
            mx = m[0]
            d = jnp.full((16,), 1.0, jnp.float32)
            for i in range(1, KK):
                d = d + jnp.exp(m[i] - mx)
            recip = 1.0 / d
            thr = m[KK - 1]
            for j in range(N_EXP):
                x = xbuf[j, pl.ds(t0, 16)]
                obuf[j, pl.ds(t0, 16)] = jnp.where(
                    x >= thr, jnp.exp(x - mx) * recip, 0.0
                )

    pending_in = [None] * NCHUNK
    pending_out = [None] * NCHUNK
    pending_in[0] = start_in(0, 0)
    for c in range(NCHUNK):
        b = c & 1
        if c + 1 < NCHUNK:
            pending_in[c + 1] = start_in(c + 1, 1 - b)
        pending_in[c].wait()
        if c >= 2:
            pending_out[c - 2].wait()
        compute_chunk(xbufs[b], obufs[b])
        pending_out[c] = pltpu.async_copy(
            obufs[b], o_hbm.at[:, pl.ds(base + c * TCHUNK, TCHUNK)], souts[b]
        )
    pending_out[NCHUNK - 2].wait()
    pending_out[NCHUNK - 1].wait()


@jax.jit
def kernel(logits):
    xt = logits.T
    mesh = plsc.VectorSubcoreMesh(core_axis_name="c", subcore_axis_name="s")
    out_t = pl.kernel(
        _body,
        out_type=jax.ShapeDtypeStruct((N_EXP, N_TOK), jnp.float32),
        mesh=mesh,
        scratch_types=[pltpu.VMEM((N_EXP, TCHUNK), jnp.float32)] * 4
        + [pltpu.SemaphoreType.DMA] * 4,
        compiler_params=pltpu.CompilerParams(needs_layout_passes=False),
    )(xt)
    return out_t.T


# speedup vs baseline: 1.0621x; 1.0621x over previous
import jax
import jax.numpy as jnp
from jax import lax
from jax.experimental import pallas as pl
from jax.experimental.pallas import tpu as pltpu
from jax.experimental.pallas import tpu_sc as plsc

N_TOK = 32768
N_EXP = 64
KK = 8
NUM_CORES = 2
NUM_SUBCORES = 16
NW = NUM_CORES * NUM_SUBCORES
TOK_PER_W = N_TOK // NW
TCHUNK = 256
NCHUNK = TOK_PER_W // TCHUNK
NGROUP = TCHUNK // 16
NEG = -3.0e38


def _body(x_hbm, o_hbm, xb0, xb1, ob0, ob1, si0, si1, so0, so1):
    wid = lax.axis_index("s") * NUM_CORES + lax.axis_index("c")
    base = wid * TOK_PER_W
    xbufs, obufs, sins, souts = (xb0, xb1), (ob0, ob1), (si0, si1), (so0, so1)

    def start_in(c, b):
        return pltpu.async_copy(
            x_hbm.at[:, pl.ds(base + c * TCHUNK, TCHUNK)], xbufs[b], sins[b]
        )

    OEMS8 = [(0, 1), (2, 3), (4, 5), (6, 7), (0, 2), (1, 3), (4, 6), (5, 7),
             (1, 2), (5, 6), (0, 4), (1, 5), (2, 6), (3, 7), (2, 4), (3, 5),
             (1, 2), (3, 4), (5, 6)]
    BITONIC8 = [(0, 4), (1, 5), (2, 6), (3, 7), (0, 2), (1, 3), (4, 6),
                (5, 7), (0, 1), (2, 3), (4, 5), (6, 7)]

    def _net(m, pairs):
        for i, j in pairs:
            hi = jnp.maximum(m[i], m[j])
            lo = jnp.minimum(m[i], m[j])
            m[i] = hi
            m[j] = lo

    def _merge8(a, b):
        t = [jnp.maximum(a[i], b[7 - i]) for i in range(8)]
        _net(t, BITONIC8)
        return t

    def compute_chunk(xbuf, obuf):
        @plsc.parallel_loop(0, NGROUP, step=1)
        def group_body(g):
            t0 = g * 16
            grps = []
            for q in range(8):
                m = [xbuf[q * 8 + i, pl.ds(t0, 16)] for i in range(8)]
                _net(m, OEMS8)
                grps.append(m)
            l1 = [_merge8(grps[0], grps[1]), _merge8(grps[2], grps[3]),
                  _merge8(grps[4], grps[5]), _merge8(grps[6], grps[7])]
            l2 = [_merge8(l1[0], l1[1]), _merge8(l1[2], l1[3])]
            t = [jnp.maximum(l2[0][i], l2[1][7 - i]) for i in range(8)]
            mx = jnp.maximum(
                jnp.maximum(jnp.maximum(t[0], t[1]), jnp.maximum(t[2], t[3])),
                jnp.maximum(jnp.maximum(t[4], t[5]), jnp.maximum(t[6], t[7])),
            )
            thr = jnp.minimum(
                jnp.minimum(jnp.minimum(t[0], t[1]), jnp.minimum(t[2], t[3])),
                jnp.minimum(jnp.minimum(t[4], t[5]), jnp.minimum(t[6], t[7])),
            )
            e = [jnp.exp(t[i] - mx) for i in range(8)]
            d = ((e[0] + e[1]) + (e[2] + e[3])) + ((e[4] + e[5]) + (e[6] + e[7]))
            recip = 1.0 / d
            for j in range(N_EXP):
                x = xbuf[j, pl.ds(t0, 16)]
                obuf[j, pl.ds(t0, 16)] = jnp.where(
                    x >= thr, jnp.exp(x - mx) * recip, 0.0
                )

    pending_in = [None] * NCHUNK
    pending_out = [None] * NCHUNK
    pending_in[0] = start_in(0, 0)
    for c in range(NCHUNK):
        b = c & 1
        if c + 1 < NCHUNK:
            pending_in[c + 1] = start_in(c + 1, 1 - b)
        pending_in[c].wait()
        if c >= 2:
            pending_out[c - 2].wait()
        compute_chunk(xbufs[b], obufs[b])
        pending_out[c] = pltpu.async_copy(
            obufs[b], o_hbm.at[:, pl.ds(base + c * TCHUNK, TCHUNK)], souts[b]
        )
    pending_out[NCHUNK - 2].wait()
    pending_out[NCHUNK - 1].wait()


@jax.jit
def kernel(logits):
    xt = logits.T
    mesh = plsc.VectorSubcoreMesh(core_axis_name="c", subcore_axis_name="s")
    out_t = pl.kernel(
        _body,
        out_type=jax.ShapeDtypeStruct((N_EXP, N_TOK), jnp.float32),
        mesh=mesh,
        scratch_types=[pltpu.VMEM((N_EXP, TCHUNK), jnp.float32)] * 4
        + [pltpu.SemaphoreType.DMA] * 4,
        compiler_params=pltpu.CompilerParams(needs_layout_passes=False),
    )(xt)
    return out_t.T
